# baseline (speedup 1.0000x reference)
import jax
import jax.numpy as jnp
from jax import lax
from jax.experimental import pallas as pl
from jax.experimental.pallas import tpu as pltpu
from jax.experimental.pallas import tpu_sc as plsc

NUM_ROWS = 100000
DIM = 128

NC = 2
NS = 16
NW = NC * NS

B_TOTAL = 4096 * 200
B_PER_W = B_TOTAL // NW
CHUNK = 128
N_CHUNKS = B_PER_W // CHUNK
NBUF = 4
N_PAIRS = N_CHUNKS // 2


def _body(ids_hbm, w_hbm, out_hbm, idx2, rows, s0, s1, s2, s3):
    sems = (s0, s1, s2, s3)
    wid = lax.axis_index("s") * NC + lax.axis_index("c")
    cbase = wid * N_CHUNKS

    pltpu.sync_copy(ids_hbm.at[wid], idx2)

    def fire_gather(c, b):
        pltpu.async_copy(w_hbm.at[idx2.at[c]], rows.at[b], sems[b])

    def wait_gather(b):
        pltpu.make_async_copy(w_hbm.at[idx2.at[0]], rows.at[b], sems[b]).wait()

    def pair(p, c0):
        wait_gather(c0)
        wait_gather(c0 + 1)
        pltpu.sync_copy(rows.at[pl.ds(c0, 2)],
                        out_hbm.at[pl.ds(cbase + 2 * p, 2)])
        g = 2 * p + NBUF
        fire_gather(g, c0)
        fire_gather(g + 1, c0 + 1)

    for b in range(NBUF):
        fire_gather(b, b)

    def outer(po, carry):
        pair(2 * po, 0)
        pair(2 * po + 1, 2)
        return carry

    lax.fori_loop(0, N_PAIRS // 2 - 1, outer, 0)
    for (p, c0) in ((N_PAIRS - 2, 0), (N_PAIRS - 1, 2)):
        wait_gather(c0)
        wait_gather(c0 + 1)
        pltpu.sync_copy(rows.at[pl.ds(c0, 2)],
                        out_hbm.at[pl.ds(cbase + 2 * p, 2)])


@jax.jit
def _run(ids3, weight):
    f = pl.kernel(
        _body,
        out_type=jax.ShapeDtypeStruct((B_TOTAL // CHUNK, CHUNK, DIM), jnp.float32),
        mesh=plsc.VectorSubcoreMesh(core_axis_name="c", subcore_axis_name="s"),
        scratch_types=[
            pltpu.VMEM((N_CHUNKS, CHUNK), jnp.int32),
            pltpu.VMEM((NBUF, CHUNK, DIM), jnp.float32),
        ] + [pltpu.SemaphoreType.DMA] * NBUF,
    )
    return f(ids3, weight)


def kernel(ids, weight):
    ids3 = ids.reshape(NW, N_CHUNKS, CHUNK).astype(jnp.int32)
    out = _run(ids3, weight)
    return out.reshape(ids.shape[0], ids.shape[1], DIM)
